# baseline (device time: 93935 ns/iter reference)
import jax
import jax.numpy as jnp
from jax import lax
from jax.experimental import pallas as pl
from jax.experimental.pallas import tpu as pltpu

N_DEV = 16
SQ = 512
D_MODEL = 1024
SKV = 2048
H_LOCAL = 8
GQA = 4
KV_LOCAL = H_LOCAL // GQA
DH = 128
SCALE = 0.08838834764831843

RS_MASKS = [1, 4, 2, 8]
AG_MASKS = [8, 2, 4, 1]
RS_ROWS = [256, 128, 64, 32]
AG_ROWS = [32, 64, 128, 256]


def _bit(m, mask):
    return lax.bitwise_and(lax.shift_right_logical(m, mask.bit_length() - 1), 1)


def kernel(x, Wq, Wo, K_ext, V_ext):
    def body(x_ref, wq_ref, wo_ref, kext_ref, vext_ref, out_ref,
             kbuf, vbuf, kv_sems,
             rs0, rs1, rs2, rs3, ag0, ag1, ag2, ag3,
             send_sems, recv_sems):
        m = lax.axis_index("i")

        copies = []
        for j in range(KV_LOCAL):
            h = m * KV_LOCAL + j
            ck = pltpu.make_async_copy(
                kext_ref.at[0, :, h, :], kbuf.at[j], kv_sems.at[2 * j])
            cv = pltpu.make_async_copy(
                vext_ref.at[0, :, h, :], vbuf.at[j], kv_sems.at[2 * j + 1])
            ck.start()
            cv.start()
            copies += [ck, cv]

        q = jnp.dot(x_ref[:], wq_ref[:], preferred_element_type=jnp.float32)

        for c in copies:
            c.wait()

        outs = []
        for h in range(H_LOCAL):
            qh = q[:, h * DH:(h + 1) * DH]
            kv = h // GQA
            s = lax.dot_general(
                qh, kbuf[kv],
                (((1,), (1,)), ((), ())),
                preferred_element_type=jnp.float32,
            ) * SCALE
            mx = jnp.max(s, axis=1, keepdims=True)
            p = jnp.exp(s - mx)
            l = jnp.sum(p, axis=1, keepdims=True)
            oh = jnp.dot(p, vbuf[kv], preferred_element_type=jnp.float32) / l
            outs.append(oh)
        attn = jnp.concatenate(outs, axis=1)
        out_ref[:] = jnp.dot(attn, wo_ref[:],
                             preferred_element_type=jnp.float32)

        rs_bufs = [rs0, rs1, rs2, rs3]
        ag_bufs = [ag0, ag1, ag2, ag3]

        lo = jnp.int32(0)
        for k, mask in enumerate(RS_MASKS):
            half = RS_ROWS[k]
            partner = lax.bitwise_xor(m, mask)
            bit = _bit(m, mask)
            send_lo = pl.multiple_of(lo + (1 - bit) * half, 32)
            keep_lo = pl.multiple_of(lo + bit * half, 32)
            rdma = pltpu.make_async_remote_copy(
                src_ref=out_ref.at[pl.ds(send_lo, half), :],
                dst_ref=rs_bufs[k],
                send_sem=send_sems.at[k],
                recv_sem=recv_sems.at[k],
                device_id=(partner,),
                device_id_type=pl.DeviceIdType.MESH,
            )
            rdma.start()
            rdma.wait()
            out_ref[pl.ds(keep_lo, half), :] = (
                out_ref[pl.ds(keep_lo, half), :] + rs_bufs[k][:]
            )
            lo = keep_lo

        for k, mask in enumerate(AG_MASKS):
            cur = AG_ROWS[k]
            partner = lax.bitwise_xor(m, mask)
            bit = _bit(m, mask)
            partner_lo = pl.multiple_of(lo + (1 - 2 * bit) * cur, 32)
            rdma = pltpu.make_async_remote_copy(
                src_ref=out_ref.at[pl.ds(pl.multiple_of(lo, 32), cur), :],
                dst_ref=ag_bufs[k],
                send_sem=send_sems.at[4 + k],
                recv_sem=recv_sems.at[4 + k],
                device_id=(partner,),
                device_id_type=pl.DeviceIdType.MESH,
            )
            rdma.start()
            rdma.wait()
            out_ref[pl.ds(partner_lo, cur), :] = ag_bufs[k][:]
            lo = jnp.minimum(lo, partner_lo)

    out = pl.pallas_call(
        body,
        out_shape=jax.ShapeDtypeStruct((SQ, D_MODEL), jnp.float32),
        in_specs=[
            pl.BlockSpec(memory_space=pltpu.VMEM),
            pl.BlockSpec(memory_space=pltpu.VMEM),
            pl.BlockSpec(memory_space=pltpu.VMEM),
            pl.BlockSpec(memory_space=pl.ANY),
            pl.BlockSpec(memory_space=pl.ANY),
        ],
        out_specs=pl.BlockSpec(memory_space=pltpu.VMEM),
        scratch_shapes=[
            pltpu.VMEM((KV_LOCAL, SKV, DH), jnp.float32),
            pltpu.VMEM((KV_LOCAL, SKV, DH), jnp.float32),
            pltpu.SemaphoreType.DMA((2 * KV_LOCAL,)),
            pltpu.VMEM((RS_ROWS[0], D_MODEL), jnp.float32),
            pltpu.VMEM((RS_ROWS[1], D_MODEL), jnp.float32),
            pltpu.VMEM((RS_ROWS[2], D_MODEL), jnp.float32),
            pltpu.VMEM((RS_ROWS[3], D_MODEL), jnp.float32),
            pltpu.VMEM((AG_ROWS[0], D_MODEL), jnp.float32),
            pltpu.VMEM((AG_ROWS[1], D_MODEL), jnp.float32),
            pltpu.VMEM((AG_ROWS[2], D_MODEL), jnp.float32),
            pltpu.VMEM((AG_ROWS[3], D_MODEL), jnp.float32),
            pltpu.SemaphoreType.DMA((8,)),
            pltpu.SemaphoreType.DMA((8,)),
        ],
        compiler_params=pltpu.CompilerParams(
            vmem_limit_bytes=96 * 1024 * 1024,
        ),
    )(x[0], Wq, Wo, K_ext, V_ext)
    return out[None]


# device time: 71688 ns/iter; 1.3103x vs baseline; 1.3103x over previous
import jax
import jax.numpy as jnp
from jax import lax
from jax.experimental import pallas as pl
from jax.experimental.pallas import tpu as pltpu

N_DEV = 16
SQ = 512
D_MODEL = 1024
SKV = 2048
H_LOCAL = 8
GQA = 4
KV_LOCAL = H_LOCAL // GQA
DH = 128
SCALE = 0.08838834764831843

RS_MASKS = [1, 4, 2, 8]
AG_MASKS = [8, 2, 4, 1]
RS_ROWS = [256, 128, 64, 32]
AG_ROWS = [32, 64, 128, 256]


def _bit(m, mask):
    return lax.bitwise_and(lax.shift_right_logical(m, mask.bit_length() - 1), 1)


def kernel(x, Wq, Wo, K_ext, V_ext):
    def body(x_ref, wq_ref, wo_ref, kext_ref, vext_ref, out_ref,
             kbuf, vbuf, kv_sems, sendb,
             rs0, rs1, rs2, rs3, ag0, ag1, ag2, ag3,
             send_sems, recv_sems):
        m = lax.axis_index("i")

        copies = []
        for j in range(KV_LOCAL):
            h = m * KV_LOCAL + j
            ck = pltpu.make_async_copy(
                kext_ref.at[0, :, h, :], kbuf.at[j], kv_sems.at[2 * j])
            cv = pltpu.make_async_copy(
                vext_ref.at[0, :, h, :], vbuf.at[j], kv_sems.at[2 * j + 1])
            ck.start()
            cv.start()
            copies += [ck, cv]

        q = jnp.dot(x_ref[:], wq_ref[:], preferred_element_type=jnp.float32)

        for c in copies:
            c.wait()

        outs = []
        for h in range(H_LOCAL):
            qh = q[:, h * DH:(h + 1) * DH]
            kv = h // GQA
            s = lax.dot_general(
                qh, kbuf[kv],
                (((1,), (1,)), ((), ())),
                preferred_element_type=jnp.float32,
            ) * SCALE
            mx = jnp.max(s, axis=1, keepdims=True)
            p = jnp.exp(s - mx)
            l = jnp.sum(p, axis=1, keepdims=True)
            oh = jnp.dot(p, vbuf[kv], preferred_element_type=jnp.float32) / l
            outs.append(oh)
        attn = jnp.concatenate(outs, axis=1)
        out_ref[:] = jnp.dot(attn, wo_ref[:],
                             preferred_element_type=jnp.float32)

        rs_bufs = [rs0, rs1, rs2, rs3]
        ag_bufs = [ag0, ag1, ag2, ag3]

        lo = jnp.int32(0)
        for k, mask in enumerate(RS_MASKS):
            half = RS_ROWS[k]
            partner = lax.bitwise_xor(m, mask)
            bit = _bit(m, mask)
            send_lo = pl.multiple_of(lo + (1 - bit) * half, 32)
            keep_lo = pl.multiple_of(lo + bit * half, 32)
            sendb[pl.ds(0, half), :] = out_ref[pl.ds(send_lo, half), :].astype(
                jnp.bfloat16)
            rdma = pltpu.make_async_remote_copy(
                src_ref=sendb.at[pl.ds(0, half), :],
                dst_ref=rs_bufs[k],
                send_sem=send_sems.at[k],
                recv_sem=recv_sems.at[k],
                device_id=(partner,),
                device_id_type=pl.DeviceIdType.MESH,
            )
            rdma.start()
            rdma.wait()
            out_ref[pl.ds(keep_lo, half), :] = (
                out_ref[pl.ds(keep_lo, half), :]
                + rs_bufs[k][:].astype(jnp.float32)
            )
            lo = keep_lo

        for k, mask in enumerate(AG_MASKS):
            cur = AG_ROWS[k]
            partner = lax.bitwise_xor(m, mask)
            bit = _bit(m, mask)
            partner_lo = pl.multiple_of(lo + (1 - 2 * bit) * cur, 32)
            sendb[pl.ds(0, cur), :] = out_ref[
                pl.ds(pl.multiple_of(lo, 32), cur), :].astype(jnp.bfloat16)
            rdma = pltpu.make_async_remote_copy(
                src_ref=sendb.at[pl.ds(0, cur), :],
                dst_ref=ag_bufs[k],
                send_sem=send_sems.at[4 + k],
                recv_sem=recv_sems.at[4 + k],
                device_id=(partner,),
                device_id_type=pl.DeviceIdType.MESH,
            )
            rdma.start()
            rdma.wait()
            out_ref[pl.ds(partner_lo, cur), :] = ag_bufs[k][:].astype(
                jnp.float32)
            lo = jnp.minimum(lo, partner_lo)

    out = pl.pallas_call(
        body,
        out_shape=jax.ShapeDtypeStruct((SQ, D_MODEL), jnp.float32),
        in_specs=[
            pl.BlockSpec(memory_space=pltpu.VMEM),
            pl.BlockSpec(memory_space=pltpu.VMEM),
            pl.BlockSpec(memory_space=pltpu.VMEM),
            pl.BlockSpec(memory_space=pl.ANY),
            pl.BlockSpec(memory_space=pl.ANY),
        ],
        out_specs=pl.BlockSpec(memory_space=pltpu.VMEM),
        scratch_shapes=[
            pltpu.VMEM((KV_LOCAL, SKV, DH), jnp.float32),
            pltpu.VMEM((KV_LOCAL, SKV, DH), jnp.float32),
            pltpu.SemaphoreType.DMA((2 * KV_LOCAL,)),
            pltpu.VMEM((RS_ROWS[0], D_MODEL), jnp.bfloat16),
            pltpu.VMEM((RS_ROWS[0], D_MODEL), jnp.bfloat16),
            pltpu.VMEM((RS_ROWS[1], D_MODEL), jnp.bfloat16),
            pltpu.VMEM((RS_ROWS[2], D_MODEL), jnp.bfloat16),
            pltpu.VMEM((RS_ROWS[3], D_MODEL), jnp.bfloat16),
            pltpu.VMEM((AG_ROWS[0], D_MODEL), jnp.bfloat16),
            pltpu.VMEM((AG_ROWS[1], D_MODEL), jnp.bfloat16),
            pltpu.VMEM((AG_ROWS[2], D_MODEL), jnp.bfloat16),
            pltpu.VMEM((AG_ROWS[3], D_MODEL), jnp.bfloat16),
            pltpu.SemaphoreType.DMA((8,)),
            pltpu.SemaphoreType.DMA((8,)),
        ],
        compiler_params=pltpu.CompilerParams(
            vmem_limit_bytes=96 * 1024 * 1024,
        ),
    )(x[0], Wq, Wo, K_ext, V_ext)
    return out[None]


# device time: 58558 ns/iter; 1.6041x vs baseline; 1.2242x over previous
import jax
import jax.numpy as jnp
from jax import lax
from jax.experimental import pallas as pl
from jax.experimental.pallas import tpu as pltpu

N_DEV = 16
SQ = 512
D_MODEL = 1024
SKV = 2048
H_LOCAL = 8
GQA = 4
KV_LOCAL = H_LOCAL // GQA
DH = 128
SCALE = 0.08838834764831843

CHUNK = SQ // N_DEV


def kernel(x, Wq, Wo, K_ext, V_ext):
    def body(x_ref, wq_ref, wo_ref, kext_ref, vext_ref, out_ref,
             kbuf, vbuf, kv_sems, sendb, agb, rs_recv, ag_recv,
             rs_send_sems, rs_recv_sems, ag_send_sems, ag_recv_sems):
        m = lax.axis_index("i")

        copies = []
        for j in range(KV_LOCAL):
            h = m * KV_LOCAL + j
            ck = pltpu.make_async_copy(
                kext_ref.at[0, :, h, :], kbuf.at[j], kv_sems.at[2 * j])
            cv = pltpu.make_async_copy(
                vext_ref.at[0, :, h, :], vbuf.at[j], kv_sems.at[2 * j + 1])
            ck.start()
            cv.start()
            copies += [ck, cv]

        q = jnp.dot(x_ref[:], wq_ref[:], preferred_element_type=jnp.float32)

        for c in copies:
            c.wait()

        outs = []
        for h in range(H_LOCAL):
            qh = q[:, h * DH:(h + 1) * DH]
            kv = h // GQA
            s = lax.dot_general(
                qh, kbuf[kv],
                (((1,), (1,)), ((), ())),
                preferred_element_type=jnp.float32,
            ) * SCALE
            mx = jnp.max(s, axis=1, keepdims=True)
            p = jnp.exp(s - mx)
            l = jnp.sum(p, axis=1, keepdims=True)
            oh = jnp.dot(p, vbuf[kv], preferred_element_type=jnp.float32) / l
            outs.append(oh)
        attn = jnp.concatenate(outs, axis=1)
        out_ref[:] = jnp.dot(attn, wo_ref[:],
                             preferred_element_type=jnp.float32)

        sendb[:] = out_ref[:].astype(jnp.bfloat16)
        rs_descs = []
        for t in range(1, N_DEV):
            d = lax.rem(m + t, N_DEV)
            rdma = pltpu.make_async_remote_copy(
                src_ref=sendb.at[pl.ds(pl.multiple_of(d * CHUNK, 32), CHUNK), :],
                dst_ref=rs_recv.at[m],
                send_sem=rs_send_sems.at[d],
                recv_sem=rs_recv_sems.at[m],
                device_id=(d,),
                device_id_type=pl.DeviceIdType.MESH,
            )
            rdma.start()
            rs_descs.append(rdma)

        my_lo = pl.multiple_of(m * CHUNK, 32)
        acc = out_ref[pl.ds(my_lo, CHUNK), :]
        for t in range(1, N_DEV):
            j = lax.rem(m + t, N_DEV)
            recv = pltpu.make_async_remote_copy(
                src_ref=sendb.at[pl.ds(0, CHUNK), :],
                dst_ref=rs_recv.at[j],
                send_sem=rs_send_sems.at[0],
                recv_sem=rs_recv_sems.at[j],
                device_id=(j,),
                device_id_type=pl.DeviceIdType.MESH,
            )
            recv.wait_recv()
            acc = acc + rs_recv[j].astype(jnp.float32)
        out_ref[pl.ds(my_lo, CHUNK), :] = acc

        agb[:] = acc.astype(jnp.bfloat16)
        ag_descs = []
        for t in range(1, N_DEV):
            d = lax.rem(m + t, N_DEV)
            rdma = pltpu.make_async_remote_copy(
                src_ref=agb,
                dst_ref=ag_recv.at[m],
                send_sem=ag_send_sems.at[d],
                recv_sem=ag_recv_sems.at[m],
                device_id=(d,),
                device_id_type=pl.DeviceIdType.MESH,
            )
            rdma.start()
            ag_descs.append(rdma)

        for t in range(1, N_DEV):
            j = lax.rem(m + t, N_DEV)
            recv = pltpu.make_async_remote_copy(
                src_ref=agb,
                dst_ref=ag_recv.at[j],
                send_sem=ag_send_sems.at[0],
                recv_sem=ag_recv_sems.at[j],
                device_id=(j,),
                device_id_type=pl.DeviceIdType.MESH,
            )
            recv.wait_recv()
            out_ref[pl.ds(pl.multiple_of(j * CHUNK, 32), CHUNK), :] = (
                ag_recv[j].astype(jnp.float32)
            )

        for rdma in rs_descs + ag_descs:
            rdma.wait_send()

    out = pl.pallas_call(
        body,
        out_shape=jax.ShapeDtypeStruct((SQ, D_MODEL), jnp.float32),
        in_specs=[
            pl.BlockSpec(memory_space=pltpu.VMEM),
            pl.BlockSpec(memory_space=pltpu.VMEM),
            pl.BlockSpec(memory_space=pltpu.VMEM),
            pl.BlockSpec(memory_space=pl.ANY),
            pl.BlockSpec(memory_space=pl.ANY),
        ],
        out_specs=pl.BlockSpec(memory_space=pltpu.VMEM),
        scratch_shapes=[
            pltpu.VMEM((KV_LOCAL, SKV, DH), jnp.float32),
            pltpu.VMEM((KV_LOCAL, SKV, DH), jnp.float32),
            pltpu.SemaphoreType.DMA((2 * KV_LOCAL,)),
            pltpu.VMEM((SQ, D_MODEL), jnp.bfloat16),
            pltpu.VMEM((CHUNK, D_MODEL), jnp.bfloat16),
            pltpu.VMEM((N_DEV, CHUNK, D_MODEL), jnp.bfloat16),
            pltpu.VMEM((N_DEV, CHUNK, D_MODEL), jnp.bfloat16),
            pltpu.SemaphoreType.DMA((N_DEV,)),
            pltpu.SemaphoreType.DMA((N_DEV,)),
            pltpu.SemaphoreType.DMA((N_DEV,)),
            pltpu.SemaphoreType.DMA((N_DEV,)),
        ],
        compiler_params=pltpu.CompilerParams(
            vmem_limit_bytes=96 * 1024 * 1024,
        ),
    )(x[0], Wq, Wo, K_ext, V_ext)
    return out[None]
